# baseline (speedup 1.0000x reference)
import functools

import jax
import jax.numpy as jnp
from jax import lax
from jax.experimental import pallas as pl
from jax.experimental.pallas import tpu as pltpu
from jax.experimental.pallas import tpu_sc as plsc

BETA = 0.8
LAMBDA_P = 1.0
LAMBDA_N = 0.1

N = 32 * 512 * 512
NC = 2
NS = 16
NW = NC * NS
PER_W = N // NW
CHUNK = 16384
NCHUNK = PER_W // CHUNK
NBUF = 2
LANES = 16

_mesh = plsc.VectorSubcoreMesh(core_axis_name="c", subcore_axis_name="s")


@functools.partial(
    pl.kernel,
    mesh=_mesh,
    out_type=jax.ShapeDtypeStruct((NW, 8, LANES), jnp.float32),
    scratch_types=[
        pltpu.VMEM((NBUF, CHUNK), jnp.float32),
        pltpu.VMEM((NBUF, CHUNK), jnp.int32),
        pltpu.VMEM((8, LANES), jnp.float32),
        pltpu.SemaphoreType.DMA,
        pltpu.SemaphoreType.DMA,
        pltpu.SemaphoreType.DMA,
        pltpu.SemaphoreType.DMA,
    ],
)
def _sc_partials(r_hbm, l_hbm, out_hbm, r_buf, l_buf, stage,
                 sem_r0, sem_r1, sem_l0, sem_l1):
    wid = lax.axis_index("s") * NC + lax.axis_index("c")
    base = wid * PER_W
    sem_r = (sem_r0, sem_r1)
    sem_l = (sem_l0, sem_l1)

    def start(k, b):
        hr = pltpu.async_copy(
            r_hbm.at[pl.ds(base + k * CHUNK, CHUNK)], r_buf.at[b], sem_r[b])
        hl = pltpu.async_copy(
            l_hbm.at[pl.ds(base + k * CHUNK, CHUNK)], l_buf.at[b], sem_l[b])
        return hr, hl

    def chunk_sums(b, carry):
        def step(i, c):
            n, sr, srp, sr2, sr2p = c
            off = i * LANES
            r = r_buf[b, pl.ds(off, LANES)]
            lf = l_buf[b, pl.ds(off, LANES)].astype(jnp.float32)
            r2 = r * r
            return (n + lf, sr + r, srp + r * lf, sr2 + r2, sr2p + r2 * lf)
        return lax.fori_loop(0, CHUNK // LANES, step, carry)

    z = jnp.zeros((LANES,), jnp.float32)
    carry = (z, z, z, z, z)
    pend = [None, None]
    pend[0] = start(0, 0)
    for k in range(NCHUNK):
        if k + 1 < NCHUNK:
            pend[(k + 1) % NBUF] = start(k + 1, (k + 1) % NBUF)
        hr, hl = pend[k % NBUF]
        hr.wait()
        hl.wait()
        carry = chunk_sums(k % NBUF, carry)

    n, sr, srp, sr2, sr2p = carry
    stage[0, :] = n
    stage[1, :] = sr
    stage[2, :] = srp
    stage[3, :] = sr2
    stage[4, :] = sr2p
    stage[5, :] = z
    stage[6, :] = z
    stage[7, :] = z
    pltpu.sync_copy(stage, out_hbm.at[wid])


def _fin_body(p_ref, o_ref):
    x = p_ref[...]
    n_pos = jnp.sum(x[:, 0:16])
    s_r = jnp.sum(x[:, 16:32])
    s_rp = jnp.sum(x[:, 32:48])
    s_r2 = jnp.sum(x[:, 48:64])
    s_r2p = jnp.sum(x[:, 64:80])
    n_neg = float(N) - n_pos
    s_rn = s_r - s_rp
    s_r2n = s_r2 - s_r2p
    mean_pos = s_rp / n_pos
    mean_neg = s_rn / n_neg
    var_pos = (s_r2p - s_rp * mean_pos) / (n_pos - 1.0)
    var_neg = (s_r2n - s_rn * mean_neg) / (n_neg - 1.0)
    loss = jnp.maximum(BETA - mean_pos, 0.0)
    loss = loss + LAMBDA_N * var_pos + mean_neg + LAMBDA_P * var_neg
    o_ref[0, 0] = loss


_finalize = pl.pallas_call(
    _fin_body,
    out_shape=jax.ShapeDtypeStruct((1, 1), jnp.float32),
    out_specs=pl.BlockSpec(memory_space=pltpu.SMEM),
)


def kernel(residues, pixel_level_labels):
    r = residues.reshape(-1)
    l = pixel_level_labels.reshape(-1)
    partials = _sc_partials(r, l)
    loss = _finalize(partials.reshape(NW, 8 * LANES))
    return loss.reshape(1)


# speedup vs baseline: 1.2036x; 1.2036x over previous
import functools

import jax
import jax.numpy as jnp
from jax import lax
from jax.experimental import pallas as pl
from jax.experimental.pallas import tpu as pltpu
from jax.experimental.pallas import tpu_sc as plsc

BETA = 0.8
LAMBDA_P = 1.0
LAMBDA_N = 0.1

N = 32 * 512 * 512
NC = 2
NS = 16
NW = NC * NS
PER_W = N // NW
CHUNK = 16384
NCHUNK = PER_W // CHUNK
NBUF = 2
LANES = 16

_mesh = plsc.VectorSubcoreMesh(core_axis_name="c", subcore_axis_name="s")


@functools.partial(
    pl.kernel,
    mesh=_mesh,
    out_type=jax.ShapeDtypeStruct((NW, 8, LANES), jnp.float32),
    scratch_types=[
        pltpu.VMEM((NBUF, CHUNK), jnp.float32),
        pltpu.VMEM((NBUF, CHUNK), jnp.int32),
        pltpu.VMEM((8, LANES), jnp.float32),
        pltpu.SemaphoreType.DMA,
        pltpu.SemaphoreType.DMA,
        pltpu.SemaphoreType.DMA,
        pltpu.SemaphoreType.DMA,
    ],
)
def _sc_partials(r_hbm, l_hbm, out_hbm, r_buf, l_buf, stage,
                 sem_r0, sem_r1, sem_l0, sem_l1):
    wid = lax.axis_index("s") * NC + lax.axis_index("c")
    base = wid * PER_W
    sem_r = (sem_r0, sem_r1)
    sem_l = (sem_l0, sem_l1)

    def start(k, b):
        hr = pltpu.async_copy(
            r_hbm.at[pl.ds(base + k * CHUNK, CHUNK)], r_buf.at[b], sem_r[b])
        hl = pltpu.async_copy(
            l_hbm.at[pl.ds(base + k * CHUNK, CHUNK)], l_buf.at[b], sem_l[b])
        return hr, hl

    def chunk_sums(b, carry):
        def step(i, c):
            n, sr, srp, sr2, sr2p = c
            off = i * LANES
            r = r_buf[b, pl.ds(off, LANES)]
            lf = l_buf[b, pl.ds(off, LANES)].astype(jnp.float32)
            r2 = r * r
            return (n + lf, sr + r, srp + r * lf, sr2 + r2, sr2p + r2 * lf)
        return lax.fori_loop(0, CHUNK // LANES, step, carry, unroll=8)

    z = jnp.zeros((LANES,), jnp.float32)
    carry = (z, z, z, z, z)
    pend = [None, None]
    pend[0] = start(0, 0)
    for k in range(NCHUNK):
        if k + 1 < NCHUNK:
            pend[(k + 1) % NBUF] = start(k + 1, (k + 1) % NBUF)
        hr, hl = pend[k % NBUF]
        hr.wait()
        hl.wait()
        carry = chunk_sums(k % NBUF, carry)

    n, sr, srp, sr2, sr2p = carry
    stage[0, :] = n
    stage[1, :] = sr
    stage[2, :] = srp
    stage[3, :] = sr2
    stage[4, :] = sr2p
    stage[5, :] = z
    stage[6, :] = z
    stage[7, :] = z
    pltpu.sync_copy(stage, out_hbm.at[wid])


def _fin_body(p_ref, o_ref):
    x = p_ref[...]
    n_pos = jnp.sum(x[:, 0:16])
    s_r = jnp.sum(x[:, 16:32])
    s_rp = jnp.sum(x[:, 32:48])
    s_r2 = jnp.sum(x[:, 48:64])
    s_r2p = jnp.sum(x[:, 64:80])
    n_neg = float(N) - n_pos
    s_rn = s_r - s_rp
    s_r2n = s_r2 - s_r2p
    mean_pos = s_rp / n_pos
    mean_neg = s_rn / n_neg
    var_pos = (s_r2p - s_rp * mean_pos) / (n_pos - 1.0)
    var_neg = (s_r2n - s_rn * mean_neg) / (n_neg - 1.0)
    loss = jnp.maximum(BETA - mean_pos, 0.0)
    loss = loss + LAMBDA_N * var_pos + mean_neg + LAMBDA_P * var_neg
    o_ref[0, 0] = loss


_finalize = pl.pallas_call(
    _fin_body,
    out_shape=jax.ShapeDtypeStruct((1, 1), jnp.float32),
    out_specs=pl.BlockSpec(memory_space=pltpu.SMEM),
)


def kernel(residues, pixel_level_labels):
    r = residues.reshape(-1)
    l = pixel_level_labels.reshape(-1)
    partials = _sc_partials(r, l)
    loss = _finalize(partials.reshape(NW, 8 * LANES))
    return loss.reshape(1)


# speedup vs baseline: 1.7098x; 1.4205x over previous
import functools

import jax
import jax.numpy as jnp
from jax import lax
from jax.experimental import pallas as pl
from jax.experimental.pallas import tpu as pltpu
from jax.experimental.pallas import tpu_sc as plsc

BETA = 0.8
LAMBDA_P = 1.0
LAMBDA_N = 0.1

B, H, W = 32, 512, 512
N = B * H * W
NC = 2
NS = 16
NW = NC * NS
ROWS = 32
NCHUNK = H // ROWS
NBUF = 2
LANES = 16
JPR = W // LANES

_mesh = plsc.VectorSubcoreMesh(core_axis_name="c", subcore_axis_name="s")


@functools.partial(
    pl.kernel,
    mesh=_mesh,
    out_type=jax.ShapeDtypeStruct((NW, 8 * LANES), jnp.float32),
    scratch_types=[
        pltpu.VMEM((NBUF, ROWS, W), jnp.float32),
        pltpu.VMEM((NBUF, ROWS, W), jnp.int32),
        pltpu.VMEM((8 * LANES,), jnp.float32),
        pltpu.SemaphoreType.DMA,
        pltpu.SemaphoreType.DMA,
        pltpu.SemaphoreType.DMA,
        pltpu.SemaphoreType.DMA,
    ],
)
def _sc_partials(r_hbm, l_hbm, out_hbm, r_buf, l_buf, stage,
                 sem_r0, sem_r1, sem_l0, sem_l1):
    wid = lax.axis_index("s") * NC + lax.axis_index("c")
    sem_r = (sem_r0, sem_r1)
    sem_l = (sem_l0, sem_l1)

    def start(k, b):
        hr = pltpu.async_copy(
            r_hbm.at[wid, 0, pl.ds(k * ROWS, ROWS), :], r_buf.at[b], sem_r[b])
        hl = pltpu.async_copy(
            l_hbm.at[wid, 0, pl.ds(k * ROWS, ROWS), :], l_buf.at[b], sem_l[b])
        return hr, hl

    def chunk_sums(b, carry):
        def row_step(i, c):
            n, sr, srp, sr2, sr2p = c
            for j in range(JPR):
                r = r_buf[b, i, pl.ds(j * LANES, LANES)]
                lf = l_buf[b, i, pl.ds(j * LANES, LANES)].astype(jnp.float32)
                r2 = r * r
                n = n + lf
                sr = sr + r
                srp = srp + r * lf
                sr2 = sr2 + r2
                sr2p = sr2p + r2 * lf
            return (n, sr, srp, sr2, sr2p)
        return lax.fori_loop(0, ROWS, row_step, carry)

    z = jnp.zeros((LANES,), jnp.float32)
    carry = (z, z, z, z, z)
    pend = [None, None]
    pend[0] = start(0, 0)
    for k in range(NCHUNK):
        if k + 1 < NCHUNK:
            pend[(k + 1) % NBUF] = start(k + 1, (k + 1) % NBUF)
        hr, hl = pend[k % NBUF]
        hr.wait()
        hl.wait()
        carry = chunk_sums(k % NBUF, carry)

    n, sr, srp, sr2, sr2p = carry
    stage[pl.ds(0, LANES)] = n
    stage[pl.ds(16, LANES)] = sr
    stage[pl.ds(32, LANES)] = srp
    stage[pl.ds(48, LANES)] = sr2
    stage[pl.ds(64, LANES)] = sr2p
    stage[pl.ds(80, LANES)] = z
    stage[pl.ds(96, LANES)] = z
    stage[pl.ds(112, LANES)] = z
    pltpu.sync_copy(stage, out_hbm.at[wid])


def _fin_body(p_ref, o_ref):
    x = p_ref[...]
    n_pos = jnp.sum(x[:, 0:16])
    s_r = jnp.sum(x[:, 16:32])
    s_rp = jnp.sum(x[:, 32:48])
    s_r2 = jnp.sum(x[:, 48:64])
    s_r2p = jnp.sum(x[:, 64:80])
    n_neg = float(N) - n_pos
    s_rn = s_r - s_rp
    s_r2n = s_r2 - s_r2p
    mean_pos = s_rp / n_pos
    mean_neg = s_rn / n_neg
    var_pos = (s_r2p - s_rp * mean_pos) / (n_pos - 1.0)
    var_neg = (s_r2n - s_rn * mean_neg) / (n_neg - 1.0)
    loss = jnp.maximum(BETA - mean_pos, 0.0)
    loss = loss + LAMBDA_N * var_pos + mean_neg + LAMBDA_P * var_neg
    o_ref[0, 0] = loss


_finalize = pl.pallas_call(
    _fin_body,
    out_shape=jax.ShapeDtypeStruct((1, 1), jnp.float32),
    out_specs=pl.BlockSpec(memory_space=pltpu.SMEM),
)


def kernel(residues, pixel_level_labels):
    partials = _sc_partials(residues, pixel_level_labels)
    return _finalize(partials).reshape(1)


# speedup vs baseline: 2.5709x; 1.5037x over previous
import functools

import jax
import jax.numpy as jnp
from jax import lax
from jax.experimental import pallas as pl
from jax.experimental.pallas import tpu as pltpu
from jax.experimental.pallas import tpu_sc as plsc

BETA = 0.8
LAMBDA_P = 1.0
LAMBDA_N = 0.1

B, H, W = 32, 512, 512
N = B * H * W
NC = 2
NS = 16
NW = NC * NS
ROWS = 32
NCHUNK = H // ROWS
NBUF = 2
LANES = 16
JPR = W // LANES

_mesh = plsc.VectorSubcoreMesh(core_axis_name="c", subcore_axis_name="s")


@functools.partial(
    pl.kernel,
    mesh=_mesh,
    out_type=jax.ShapeDtypeStruct((NW, 8 * LANES), jnp.float32),
    scratch_types=[
        pltpu.VMEM((NBUF, ROWS, W), jnp.float32),
        pltpu.VMEM((NBUF, ROWS, W), jnp.int32),
        pltpu.VMEM((8 * LANES,), jnp.float32),
        pltpu.SemaphoreType.DMA,
        pltpu.SemaphoreType.DMA,
        pltpu.SemaphoreType.DMA,
        pltpu.SemaphoreType.DMA,
    ],
)
def _sc_partials(r_hbm, l_hbm, out_hbm, r_buf, l_buf, stage,
                 sem_r0, sem_r1, sem_l0, sem_l1):
    wid = lax.axis_index("s") * NC + lax.axis_index("c")
    sem_r = (sem_r0, sem_r1)
    sem_l = (sem_l0, sem_l1)

    def start(k, b):
        hr = pltpu.async_copy(
            r_hbm.at[wid, 0, pl.ds(k * ROWS, ROWS), :], r_buf.at[b], sem_r[b])
        hl = pltpu.async_copy(
            l_hbm.at[wid, 0, pl.ds(k * ROWS, ROWS), :], l_buf.at[b], sem_l[b])
        return hr, hl

    def chunk_sums(b, carry):
        def row_step(i, c):
            def j_step(j, c2):
                n, sr, srp, sr2, sr2p = c2
                r = r_buf[b, i, pl.ds(j * LANES, LANES)]
                lf = l_buf[b, i, pl.ds(j * LANES, LANES)].astype(jnp.float32)
                r2 = r * r
                return (n + lf, sr + r, srp + r * lf,
                        sr2 + r2, sr2p + r2 * lf)
            return lax.fori_loop(0, JPR, j_step, c, unroll=8)
        return lax.fori_loop(0, ROWS, row_step, carry)

    z = jnp.zeros((LANES,), jnp.float32)
    carry = (z, z, z, z, z)
    pend = [None, None]
    pend[0] = start(0, 0)
    for k in range(NCHUNK):
        if k + 1 < NCHUNK:
            pend[(k + 1) % NBUF] = start(k + 1, (k + 1) % NBUF)
        hr, hl = pend[k % NBUF]
        hr.wait()
        hl.wait()
        carry = chunk_sums(k % NBUF, carry)

    n, sr, srp, sr2, sr2p = carry
    stage[pl.ds(0, LANES)] = n
    stage[pl.ds(16, LANES)] = sr
    stage[pl.ds(32, LANES)] = srp
    stage[pl.ds(48, LANES)] = sr2
    stage[pl.ds(64, LANES)] = sr2p
    stage[pl.ds(80, LANES)] = z
    stage[pl.ds(96, LANES)] = z
    stage[pl.ds(112, LANES)] = z
    pltpu.sync_copy(stage, out_hbm.at[wid])


def _fin_body(p_ref, o_ref):
    x = p_ref[...]
    n_pos = jnp.sum(x[:, 0:16])
    s_r = jnp.sum(x[:, 16:32])
    s_rp = jnp.sum(x[:, 32:48])
    s_r2 = jnp.sum(x[:, 48:64])
    s_r2p = jnp.sum(x[:, 64:80])
    n_neg = float(N) - n_pos
    s_rn = s_r - s_rp
    s_r2n = s_r2 - s_r2p
    mean_pos = s_rp / n_pos
    mean_neg = s_rn / n_neg
    var_pos = (s_r2p - s_rp * mean_pos) / (n_pos - 1.0)
    var_neg = (s_r2n - s_rn * mean_neg) / (n_neg - 1.0)
    loss = jnp.maximum(BETA - mean_pos, 0.0)
    loss = loss + LAMBDA_N * var_pos + mean_neg + LAMBDA_P * var_neg
    o_ref[0, 0] = loss


_finalize = pl.pallas_call(
    _fin_body,
    out_shape=jax.ShapeDtypeStruct((1, 1), jnp.float32),
    out_specs=pl.BlockSpec(memory_space=pltpu.SMEM),
)


def kernel(residues, pixel_level_labels):
    partials = _sc_partials(residues, pixel_level_labels)
    return _finalize(partials).reshape(1)
